# baseline (speedup 1.0000x reference)
import functools

import jax
import jax.numpy as jnp
from jax import lax
from jax.experimental import pallas as pl
from jax.experimental.pallas import tpu as pltpu
from jax.experimental.pallas import tpu_sc as plsc

N = 50000
D = 128
R = 4
L = 2
E = 160000

NC = 2
NS = 16
NB = 4
W = 12544
NP = NB * W

RE = R * E
EROW = 128
QUANT = NS * EROW
PADTOT = RE + NB * QUANT
PADROWS = PADTOT // EROW
PADTOT2 = PADTOT + EROW
PKB = 8
EXT_SUB = PKB * (-(-(PADTOT) // (NC * NS * EROW * PKB)))
EXTROWS = NC * NS * EXT_SUB
EXTTOT = EXTROWS * EROW

ACC_SUB = W // NS
ZB = 56

MM_BN = 3136
FIN_BN = 2000

_GATHER_DNUMS = lax.GatherDimensionNumbers(
    offset_dims=(), collapsed_slice_dims=(0,), start_index_map=(0,))


def _mm_body(x_ref, w_ref, out_ref, *, relu):
    x = x_ref[...]
    if relu:
        x = jnp.maximum(x, 0.0)
    out_ref[...] = lax.dot_general(x, w_ref[0], (((1,), (1,)), ((), ())))


def _mm(x, w, *, relu):
    nb = NP // MM_BN
    return pl.pallas_call(
        functools.partial(_mm_body, relu=relu),
        grid=(R, nb),
        in_specs=[pl.BlockSpec((MM_BN, D), lambda r, n: (n, 0)),
                  pl.BlockSpec((1, D, D), lambda r, n: (r, 0, 0))],
        out_specs=pl.BlockSpec((MM_BN, D), lambda r, n: (r * nb + n, 0)),
        out_shape=jax.ShapeDtypeStruct((R * NP, D), jnp.float32),
    )(x, w)


def _fin_body(acc_ref, out_ref):
    x = jnp.maximum(acc_ref[...], 0.0)
    ssum = jnp.sum(x * x, axis=1, keepdims=True)
    out_ref[...] = x * (1.0 / jnp.maximum(jnp.sqrt(ssum), 1e-12))


def _finalize(acc):
    return pl.pallas_call(
        _fin_body,
        grid=(N // FIN_BN,),
        in_specs=[pl.BlockSpec((FIN_BN, D), lambda n: (n, 0))],
        out_specs=pl.BlockSpec((FIN_BN, D), lambda n: (n, 0)),
        out_shape=jax.ShapeDtypeStruct((N, D), jnp.float32),
    )(acc)


def _sc_agg_body(yt_hbm, src_hbm, dst_hbm, w_hbm, meta_hbm, out_hbm,
                 acc_sp, meta_v, idx_v, dst_v, w_v, rows_v, zero_v, sem):
    c = lax.axis_index("c")
    s = lax.axis_index("s")
    pltpu.sync_copy(meta_hbm.at[c], meta_v)
    mv = meta_v[...]
    zvec = jnp.zeros((16,), jnp.float32)
    for i in range(ZB):
        for k in range(D // 16):
            zero_v[i, 0, pl.ds(k * 16, 16)] = zvec
    for p in range(2):
        b = 2 * p + c
        start_row = mv[2 * p]
        nrows_sub = mv[2 * p + 1]

        def zbody(i, carry):
            pltpu.sync_copy(zero_v, acc_sp.at[pl.ds(s * ACC_SUB + i * ZB, ZB)])
            return carry
        lax.fori_loop(0, ACC_SUB // ZB, zbody, 0)
        plsc.subcore_barrier()

        rstart = start_row + s * nrows_sub

        def eblock(i, carry):
            row = rstart + i
            pltpu.sync_copy(src_hbm.at[pl.ds(row * EROW, EROW)], idx_v.at[0])
            pltpu.sync_copy(dst_hbm.at[pl.ds(row * EROW, EROW)], dst_v.at[0])
            pltpu.sync_copy(w_hbm.at[pl.ds(row * EROW, EROW)], w_v.at[0])
            pltpu.async_copy(yt_hbm.at[idx_v.at[0]], rows_v, sem).wait()

            def gbody(g, carry2):
                wv = w_v[0, pl.ds(g * 16, 16)]
                for i2 in range(16):
                    e = g * 16 + i2
                    ws = lax.gather(
                        wv, jnp.full((16, 1), i2, jnp.int32),
                        _GATHER_DNUMS, (1,),
                        mode=lax.GatherScatterMode.PROMISE_IN_BOUNDS)
                    for k in range(D // 16):
                        rows_v[e, 0, pl.ds(k * 16, 16)] = (
                            rows_v[e, 0, pl.ds(k * 16, 16)] * ws)
                return carry2
            lax.fori_loop(0, EROW // 16, gbody, 0)
            pltpu.sync_copy(rows_v, acc_sp.at[dst_v.at[0]], add=True)
            return carry
        lax.fori_loop(0, nrows_sub, eblock, 0)
        plsc.subcore_barrier()
        pltpu.sync_copy(
            acc_sp.at[pl.ds(s * ACC_SUB, ACC_SUB)],
            out_hbm.at[pl.ds(b * W + s * ACC_SUB, ACC_SUB)])
        if p == 0:
            plsc.subcore_barrier()


_sc_agg = functools.partial(
    pl.kernel,
    out_type=jax.ShapeDtypeStruct((NP, 1, D), jnp.float32),
    mesh=plsc.VectorSubcoreMesh(core_axis_name="c", subcore_axis_name="s",
                                num_cores=NC),
    scratch_types=[
        pltpu.VMEM_SHARED((W, 1, D), jnp.float32),
        pltpu.VMEM((16,), jnp.int32),
        pltpu.VMEM((1, EROW), jnp.int32),
        pltpu.VMEM((1, EROW), jnp.int32),
        pltpu.VMEM((1, EROW), jnp.float32),
        pltpu.VMEM((EROW, 1, D), jnp.float32),
        pltpu.VMEM((ZB, 1, D), jnp.float32),
        pltpu.SemaphoreType.DMA,
    ],
)(_sc_agg_body)


def _sc_permute_body(pos_hbm, src_hbm, dst_hbm, w_hbm,
                     osrc, odst, ow, pos_v, sv, dv, wv, sem):
    c = lax.axis_index("c")
    s = lax.axis_index("s")
    rbase = (s * NC + c) * (EXT_SUB // PKB)

    def rbody(r, carry):
        row = (rbase + r) * PKB
        pltpu.sync_copy(pos_hbm.at[pl.ds(row, PKB)], pos_v)
        pltpu.sync_copy(src_hbm.at[pl.ds(row, PKB)], sv)
        pltpu.sync_copy(dst_hbm.at[pl.ds(row, PKB)], dv)
        pltpu.sync_copy(w_hbm.at[pl.ds(row, PKB)], wv)
        descs = []
        for j in range(PKB):
            descs.append(pltpu.async_copy(
                sv.at[j, 0], osrc.at[pos_v.at[j, 0]], sem))
            descs.append(pltpu.async_copy(
                dv.at[j, 0], odst.at[pos_v.at[j, 0]], sem))
            descs.append(pltpu.async_copy(
                wv.at[j, 0], ow.at[pos_v.at[j, 0]], sem))
        for d in descs:
            d.wait()
        return carry
    lax.fori_loop(0, EXT_SUB // PKB, rbody, 0)


_sc_permute = functools.partial(
    pl.kernel,
    out_type=(jax.ShapeDtypeStruct((PADTOT2,), jnp.int32),
              jax.ShapeDtypeStruct((PADTOT2,), jnp.int32),
              jax.ShapeDtypeStruct((PADTOT2,), jnp.float32)),
    mesh=plsc.VectorSubcoreMesh(core_axis_name="c", subcore_axis_name="s",
                                num_cores=NC),
    scratch_types=[
        pltpu.VMEM((PKB, 1, EROW), jnp.int32),
        pltpu.VMEM((PKB, 1, EROW), jnp.int32),
        pltpu.VMEM((PKB, 1, EROW), jnp.int32),
        pltpu.VMEM((PKB, 1, EROW), jnp.float32),
        pltpu.SemaphoreType.DMA,
    ],
)(_sc_permute_body)


def _prep_edges(edge_index, edge_weight):
    src = edge_index[:, 1, :].astype(jnp.int32)
    dst = edge_index[:, 0, :].astype(jnp.int32)
    src2 = (src + (jnp.arange(R, dtype=jnp.int32) * NP)[:, None]).reshape(-1)
    dstf = dst.reshape(-1)
    w2 = edge_weight.reshape(-1)
    b = dstf // W
    oh = (b[:, None] == jnp.arange(NB, dtype=jnp.int32)[None, :]).astype(jnp.int32)
    csum = jnp.cumsum(oh, axis=0)
    counts = csum[-1]
    rank = jnp.sum(oh * csum, axis=1) - 1
    lb = ((counts + (QUANT - 1)) // QUANT) * QUANT
    ps = jnp.concatenate([jnp.zeros((1,), jnp.int32),
                          jnp.cumsum(lb)[:NB - 1].astype(jnp.int32)])
    pos = ps[b] + rank
    karange = jnp.arange(QUANT, dtype=jnp.int32)[None, :]
    gap = (lb - counts)[:, None]
    pad_pos = jnp.where(
        karange < gap, (ps + counts)[:, None] + karange,
        PADTOT + (karange % EROW)).reshape(-1)
    trash_pos = PADTOT + (jnp.arange(EXTTOT - RE - NB * QUANT,
                                     dtype=jnp.int32) % EROW)
    zi = jnp.zeros_like(pad_pos)
    zt = jnp.zeros_like(trash_pos)
    pos_ext = jnp.concatenate([pos, pad_pos, trash_pos])
    src_ext = jnp.concatenate([src2, zi, zt])
    dst_ext = jnp.concatenate([dstf - b * W, zi, zt])
    w_ext = jnp.concatenate([w2, zi.astype(jnp.float32),
                             zt.astype(jnp.float32)])
    ssrc, sdst, sw = _sc_permute(
        pos_ext.reshape(EXTROWS, 1, EROW), src_ext.reshape(EXTROWS, 1, EROW),
        dst_ext.reshape(EXTROWS, 1, EROW),
        w_ext.reshape(EXTROWS, 1, EROW))
    ps_row = ps // EROW
    nsub = lb // QUANT
    meta = jnp.stack([
        jnp.concatenate([jnp.stack([ps_row[c], nsub[c], ps_row[c + 2],
                                    nsub[c + 2]]),
                         jnp.zeros((12,), jnp.int32)])
        for c in range(NC)]).astype(jnp.int32)
    return ssrc, sdst, sw, meta


def kernel(edge_index, edge_weight, ent_emb, rel_trans):
    ssrc, sdst, sw, meta = _prep_edges(edge_index, edge_weight)
    emb = jnp.pad(ent_emb, ((0, NP - N), (0, 0)))
    for l in range(L):
        yt = _mm(emb, rel_trans[l], relu=(l > 0))
        emb = _sc_agg(yt.reshape(R * NP, 1, D), ssrc, sdst, sw,
                      meta).reshape(NP, D)
    return _finalize(emb)


# speedup vs baseline: 2.8928x; 2.8928x over previous
import functools

import jax
import jax.numpy as jnp
from jax import lax
from jax.experimental import pallas as pl
from jax.experimental.pallas import tpu as pltpu
from jax.experimental.pallas import tpu_sc as plsc

N = 50000
D = 128
R = 4
L = 2
E = 160000

NC = 2
NS = 16
NB = 4
W = 12544
NP = NB * W

RE = R * E
EROW = 128
QUANT = NS * EROW
PADTOT = RE + NB * QUANT
PADROWS = PADTOT // EROW
TRASH = 3072
PADTOT2 = PADTOT + TRASH
PKB = 8
EXT_SUB = PKB * (-(-(PADTOT) // (NS * EROW * PKB)))
EXTROWS = NS * EXT_SUB
EXTTOT = EXTROWS * EROW
WR_SUB = PADTOT2 // (NC * NS)

ACC_SUB = W // NS
ZB = 56

MM_BN = 3136
FIN_BN = 2000

_GATHER_DNUMS = lax.GatherDimensionNumbers(
    offset_dims=(), collapsed_slice_dims=(0,), start_index_map=(0,))


def _mm_body(x_ref, w_ref, out_ref, *, relu):
    x = x_ref[...]
    if relu:
        x = jnp.maximum(x, 0.0)
    out_ref[...] = lax.dot_general(x, w_ref[0], (((1,), (1,)), ((), ())))


def _mm(x, w, *, relu):
    nb = NP // MM_BN
    return pl.pallas_call(
        functools.partial(_mm_body, relu=relu),
        grid=(R, nb),
        in_specs=[pl.BlockSpec((MM_BN, D), lambda r, n: (n, 0)),
                  pl.BlockSpec((1, D, D), lambda r, n: (r, 0, 0))],
        out_specs=pl.BlockSpec((MM_BN, D), lambda r, n: (r * nb + n, 0)),
        out_shape=jax.ShapeDtypeStruct((R * NP, D), jnp.float32),
    )(x, w)


def _fin_body(acc_ref, out_ref):
    x = jnp.maximum(acc_ref[...], 0.0)
    ssum = jnp.sum(x * x, axis=1, keepdims=True)
    out_ref[...] = x * (1.0 / jnp.maximum(jnp.sqrt(ssum), 1e-12))


def _finalize(acc):
    return pl.pallas_call(
        _fin_body,
        grid=(N // FIN_BN,),
        in_specs=[pl.BlockSpec((FIN_BN, D), lambda n: (n, 0))],
        out_specs=pl.BlockSpec((FIN_BN, D), lambda n: (n, 0)),
        out_shape=jax.ShapeDtypeStruct((N, D), jnp.float32),
    )(acc)


def _sc_agg_body(yt_hbm, src_hbm, dst_hbm, w_hbm, meta_hbm, out_hbm,
                 acc_sp, meta_v, idx_v, dst_v, w_v, rows_v, zero_v, sem):
    c = lax.axis_index("c")
    s = lax.axis_index("s")
    pltpu.sync_copy(meta_hbm.at[c], meta_v)
    mv = meta_v[...]
    zvec = jnp.zeros((16,), jnp.float32)
    for i in range(ZB):
        for k in range(D // 16):
            zero_v[i, 0, pl.ds(k * 16, 16)] = zvec
    for p in range(2):
        b = 2 * p + c
        start_row = mv[2 * p]
        nrows_sub = mv[2 * p + 1]

        def zbody(i, carry):
            pltpu.sync_copy(zero_v, acc_sp.at[pl.ds(s * ACC_SUB + i * ZB, ZB)])
            return carry
        lax.fori_loop(0, ACC_SUB // ZB, zbody, 0)
        plsc.subcore_barrier()

        rstart = start_row + s * nrows_sub

        def eblock(i, carry):
            row = rstart + i
            pltpu.sync_copy(src_hbm.at[pl.ds(row * EROW, EROW)], idx_v.at[0])
            pltpu.sync_copy(dst_hbm.at[pl.ds(row * EROW, EROW)], dst_v.at[0])
            pltpu.sync_copy(w_hbm.at[pl.ds(row * EROW, EROW)], w_v.at[0])
            pltpu.async_copy(yt_hbm.at[idx_v.at[0]], rows_v, sem).wait()

            def gbody(g, carry2):
                wv = w_v[0, pl.ds(g * 16, 16)]
                for i2 in range(16):
                    e = g * 16 + i2
                    ws = lax.gather(
                        wv, jnp.full((16, 1), i2, jnp.int32),
                        _GATHER_DNUMS, (1,),
                        mode=lax.GatherScatterMode.PROMISE_IN_BOUNDS)
                    for k in range(D // 16):
                        rows_v[e, 0, pl.ds(k * 16, 16)] = (
                            rows_v[e, 0, pl.ds(k * 16, 16)] * ws)
                return carry2
            lax.fori_loop(0, EROW // 16, gbody, 0)
            pltpu.sync_copy(rows_v, acc_sp.at[dst_v.at[0]], add=True)
            return carry
        lax.fori_loop(0, nrows_sub, eblock, 0)
        plsc.subcore_barrier()
        pltpu.sync_copy(
            acc_sp.at[pl.ds(s * ACC_SUB, ACC_SUB)],
            out_hbm.at[pl.ds(b * W + s * ACC_SUB, ACC_SUB)])
        if p == 0:
            plsc.subcore_barrier()


_sc_agg = functools.partial(
    pl.kernel,
    out_type=jax.ShapeDtypeStruct((NP, 1, D), jnp.float32),
    mesh=plsc.VectorSubcoreMesh(core_axis_name="c", subcore_axis_name="s",
                                num_cores=NC),
    scratch_types=[
        pltpu.VMEM_SHARED((W, 1, D), jnp.float32),
        pltpu.VMEM((16,), jnp.int32),
        pltpu.VMEM((1, EROW), jnp.int32),
        pltpu.VMEM((1, EROW), jnp.int32),
        pltpu.VMEM((1, EROW), jnp.float32),
        pltpu.VMEM((EROW, 1, D), jnp.float32),
        pltpu.VMEM((ZB, 1, D), jnp.float32),
        pltpu.SemaphoreType.DMA,
    ],
)(_sc_agg_body)


def _sc_permute_body(pos_hbm, src_hbm, dst_hbm, w_hbm,
                     osrc, odst, ow, ssrc_sp, sdst_sp, sw_sp,
                     pos_v, sv, dv, wv, sem):
    c = lax.axis_index("c")
    s = lax.axis_index("s")
    rbase = s * (EXT_SUB // PKB)

    def rbody(r, carry):
        row = (rbase + r) * PKB
        pltpu.sync_copy(pos_hbm.at[pl.ds(row, PKB)], pos_v)
        pltpu.sync_copy(src_hbm.at[pl.ds(row, PKB)], sv)
        pltpu.sync_copy(dst_hbm.at[pl.ds(row, PKB)], dv)
        pltpu.sync_copy(w_hbm.at[pl.ds(row, PKB)], wv)
        descs = []
        for j in range(PKB):
            descs.append(pltpu.async_copy(
                sv.at[j, 0], ssrc_sp.at[pos_v.at[j, 0]], sem))
            descs.append(pltpu.async_copy(
                dv.at[j, 0], sdst_sp.at[pos_v.at[j, 0]], sem))
            descs.append(pltpu.async_copy(
                wv.at[j, 0], sw_sp.at[pos_v.at[j, 0]], sem))
        for d in descs:
            d.wait()
        return carry
    lax.fori_loop(0, EXT_SUB // PKB, rbody, 0)
    plsc.subcore_barrier()
    off = (c * NS + s) * WR_SUB
    pltpu.sync_copy(ssrc_sp.at[pl.ds(off, WR_SUB)], osrc.at[pl.ds(off, WR_SUB)])
    pltpu.sync_copy(sdst_sp.at[pl.ds(off, WR_SUB)], odst.at[pl.ds(off, WR_SUB)])
    pltpu.sync_copy(sw_sp.at[pl.ds(off, WR_SUB)], ow.at[pl.ds(off, WR_SUB)])


_sc_permute = functools.partial(
    pl.kernel,
    out_type=(jax.ShapeDtypeStruct((PADTOT2,), jnp.int32),
              jax.ShapeDtypeStruct((PADTOT2,), jnp.int32),
              jax.ShapeDtypeStruct((PADTOT2,), jnp.float32)),
    mesh=plsc.VectorSubcoreMesh(core_axis_name="c", subcore_axis_name="s",
                                num_cores=NC),
    scratch_types=[
        pltpu.VMEM_SHARED((PADTOT2,), jnp.int32),
        pltpu.VMEM_SHARED((PADTOT2,), jnp.int32),
        pltpu.VMEM_SHARED((PADTOT2,), jnp.float32),
        pltpu.VMEM((PKB, 1, EROW), jnp.int32),
        pltpu.VMEM((PKB, 1, EROW), jnp.int32),
        pltpu.VMEM((PKB, 1, EROW), jnp.int32),
        pltpu.VMEM((PKB, 1, EROW), jnp.float32),
        pltpu.SemaphoreType.DMA,
    ],
)(_sc_permute_body)


def _prep_edges(edge_index, edge_weight):
    src = edge_index[:, 1, :].astype(jnp.int32)
    dst = edge_index[:, 0, :].astype(jnp.int32)
    src2 = (src + (jnp.arange(R, dtype=jnp.int32) * NP)[:, None]).reshape(-1)
    dstf = dst.reshape(-1)
    w2 = edge_weight.reshape(-1)
    b = dstf // W
    oh = (b[:, None] == jnp.arange(NB, dtype=jnp.int32)[None, :]).astype(jnp.int32)
    csum = jnp.cumsum(oh, axis=0)
    counts = csum[-1]
    rank = jnp.sum(oh * csum, axis=1) - 1
    lb = ((counts + (QUANT - 1)) // QUANT) * QUANT
    ps = jnp.concatenate([jnp.zeros((1,), jnp.int32),
                          jnp.cumsum(lb)[:NB - 1].astype(jnp.int32)])
    pos = ps[b] + rank
    karange = jnp.arange(QUANT, dtype=jnp.int32)[None, :]
    gap = (lb - counts)[:, None]
    pad_pos = jnp.where(
        karange < gap, (ps + counts)[:, None] + karange,
        PADTOT + (karange % TRASH)).reshape(-1)
    trash_pos = PADTOT + (jnp.arange(EXTTOT - RE - NB * QUANT,
                                     dtype=jnp.int32) % TRASH)
    zi = jnp.zeros_like(pad_pos)
    zt = jnp.zeros_like(trash_pos)
    pos_ext = jnp.concatenate([pos, pad_pos, trash_pos])
    src_ext = jnp.concatenate([src2, zi, zt])
    dst_ext = jnp.concatenate([dstf - b * W, zi, zt])
    w_ext = jnp.concatenate([w2, zi.astype(jnp.float32),
                             zt.astype(jnp.float32)])
    ssrc, sdst, sw = _sc_permute(
        pos_ext.reshape(EXTROWS, 1, EROW), src_ext.reshape(EXTROWS, 1, EROW),
        dst_ext.reshape(EXTROWS, 1, EROW),
        w_ext.reshape(EXTROWS, 1, EROW))
    ps_row = ps // EROW
    nsub = lb // QUANT
    meta = jnp.stack([
        jnp.concatenate([jnp.stack([ps_row[c], nsub[c], ps_row[c + 2],
                                    nsub[c + 2]]),
                         jnp.zeros((12,), jnp.int32)])
        for c in range(NC)]).astype(jnp.int32)
    return ssrc, sdst, sw, meta


def kernel(edge_index, edge_weight, ent_emb, rel_trans):
    ssrc, sdst, sw, meta = _prep_edges(edge_index, edge_weight)
    emb = jnp.pad(ent_emb, ((0, NP - N), (0, 0)))
    for l in range(L):
        yt = _mm(emb, rel_trans[l], relu=(l > 0))
        emb = _sc_agg(yt.reshape(R * NP, 1, D), ssrc, sdst, sw,
                      meta).reshape(NP, D)
    return _finalize(emb)
